# baseline (speedup 1.0000x reference)
import jax
import jax.numpy as jnp
from jax import lax
from jax.experimental import pallas as pl
from jax.experimental.pallas import tpu as pltpu, tpu_sc as plsc

N_NODES = 10000
N_PAD = 10240
N_EDGES = 320000
CHUNK = 80
CPW = 125
CHUNK_B = 128
CPW_B = 79
E_PAD = 32 * CPW * CHUNK
N_GRAPHS = 64
ROWS_PER_SUB = N_PAD // 16

_HIGH = lax.Precision.HIGHEST


def _sc_mesh():
    return plsc.VectorSubcoreMesh(
        core_axis_name="c", subcore_axis_name="s", num_cores=2, num_subcores=16
    )


def _sc_hist(dst_r, ones_rows, zrows):
    def body(dst_hbm, ones_hbm, z_hbm, out_hbm, dst_v, ones_v, agg_sh, sem):
        c = lax.axis_index("c")
        s = lax.axis_index("s")
        w = s * 2 + c
        pltpu.sync_copy(dst_hbm.at[w], dst_v)
        pltpu.sync_copy(ones_hbm, ones_v)
        pltpu.sync_copy(z_hbm, agg_sh.at[pl.ds(s * ROWS_PER_SUB, ROWS_PER_SUB)])
        plsc.subcore_barrier()

        def chunk(i, carry):
            pltpu.sync_copy(ones_v, agg_sh.at[dst_v.at[i]], add=True)
            return carry

        lax.fori_loop(0, CPW_B, chunk, 0)
        plsc.subcore_barrier()
        pltpu.sync_copy(
            agg_sh.at[pl.ds(s * ROWS_PER_SUB, ROWS_PER_SUB)],
            out_hbm.at[c, pl.ds(s * ROWS_PER_SUB, ROWS_PER_SUB)],
        )

    k = pl.kernel(
        body,
        out_type=jax.ShapeDtypeStruct((2, N_PAD, 16), jnp.float32),
        mesh=_sc_mesh(),
        compiler_params=pltpu.CompilerParams(use_tc_tiling_on_sc=False),
        scratch_types=[
            pltpu.VMEM((CPW_B, CHUNK_B), jnp.int32),
            pltpu.VMEM((CHUNK_B, 16), jnp.float32),
            pltpu.VMEM_SHARED((N_PAD, 16), jnp.float32),
            pltpu.SemaphoreType.DMA,
        ],
    )
    return k(dst_r, ones_rows, zrows)


def _sc_agg(p, src_r, dst_r, zrows, feat, cpw, chunk_sz):
    epw = cpw * chunk_sz

    def body(p_hbm, src_hbm, dst_hbm, z_hbm, out_hbm, src_v, dst_v, rows_a,
             rows_b, agg_sh, gsem_a, gsem_b, ssem_a, ssem_b):
        c = lax.axis_index("c")
        s = lax.axis_index("s")
        w = s * 2 + c
        pltpu.sync_copy(src_hbm.at[pl.ds(w * epw, epw)], src_v)
        pltpu.sync_copy(dst_hbm.at[w], dst_v)
        pltpu.sync_copy(z_hbm, agg_sh.at[pl.ds(s * ROWS_PER_SUB, ROWS_PER_SUB)])
        plsc.subcore_barrier()

        def g_start(i, buf, sem):
            pltpu.async_copy(p_hbm.at[src_v.at[pl.ds(i * chunk_sz, chunk_sz)]], buf, sem)

        def g_wait(i, buf, sem):
            pltpu.make_async_copy(
                p_hbm.at[src_v.at[pl.ds(i * chunk_sz, chunk_sz)]], buf, sem
            ).wait()

        def s_start(i, buf, sem):
            pltpu.async_copy(buf, agg_sh.at[dst_v.at[i]], sem, add=True)

        def s_wait(i, buf, sem):
            pltpu.make_async_copy(buf, agg_sh.at[dst_v.at[i]], sem).wait()

        g_start(0, rows_a, gsem_a)
        g_start(1, rows_b, gsem_b)

        def step(k, carry):
            i = 2 * k
            g_wait(i, rows_a, gsem_a)
            s_start(i, rows_a, ssem_a)
            g_wait(i + 1, rows_b, gsem_b)
            s_start(i + 1, rows_b, ssem_b)
            s_wait(i, rows_a, ssem_a)
            g_start(i + 2, rows_a, gsem_a)
            s_wait(i + 1, rows_b, ssem_b)

            @pl.when(i + 3 < cpw)
            def _():
                g_start(i + 3, rows_b, gsem_b)

            return carry

        lax.fori_loop(0, (cpw - 1) // 2, step, 0)
        g_wait(cpw - 1, rows_a, gsem_a)
        s_start(cpw - 1, rows_a, ssem_a)
        s_wait(cpw - 1, rows_a, ssem_a)
        plsc.subcore_barrier()
        pltpu.sync_copy(
            agg_sh.at[pl.ds(s * ROWS_PER_SUB, ROWS_PER_SUB)],
            out_hbm.at[c, pl.ds(s * ROWS_PER_SUB, ROWS_PER_SUB)],
        )

    extra = {} if feat == 128 else dict(
        compiler_params=pltpu.CompilerParams(use_tc_tiling_on_sc=False))
    k = pl.kernel(
        body,
        out_type=jax.ShapeDtypeStruct((2, N_PAD, feat), jnp.float32),
        mesh=_sc_mesh(),
        **extra,
        scratch_types=[
            pltpu.VMEM((cpw * chunk_sz,), jnp.int32),
            pltpu.VMEM((cpw, chunk_sz), jnp.int32),
            pltpu.VMEM((chunk_sz, feat), jnp.float32),
            pltpu.VMEM((chunk_sz, feat), jnp.float32),
            pltpu.VMEM_SHARED((N_PAD, feat), jnp.float32),
            pltpu.SemaphoreType.DMA,
            pltpu.SemaphoreType.DMA,
            pltpu.SemaphoreType.DMA,
            pltpu.SemaphoreType.DMA,
        ],
    )
    return k(p, src_r, dst_r, zrows)


def _tc_scale1(dom_x, w1cat, degp):
    blk = 1000

    def body(x_ref, w_ref, deg_ref, p1_ref, dinv_ref):
        d = deg_ref[0, :, 0:1] + deg_ref[1, :, 0:1] + 1.0
        dinv = lax.rsqrt(d)
        p1_ref[...] = (
            jnp.dot(x_ref[...], w_ref[...], preferred_element_type=jnp.float32,
                    precision=_HIGH)
            * dinv
        )
        dinv_ref[...] = jnp.broadcast_to(dinv, (blk, 8))

    return pl.pallas_call(
        body,
        grid=(N_NODES // blk,),
        in_specs=[
            pl.BlockSpec((blk, 128), lambda i: (i, 0)),
            pl.BlockSpec((128, 128), lambda i: (0, 0)),
            pl.BlockSpec((2, blk, 16), lambda i: (0, i, 0)),
        ],
        out_specs=[
            pl.BlockSpec((blk, 128), lambda i: (i, 0)),
            pl.BlockSpec((blk, 8), lambda i: (i, 0)),
        ],
        out_shape=[
            jax.ShapeDtypeStruct((N_NODES, 128), jnp.float32),
            jax.ShapeDtypeStruct((N_NODES, 8), jnp.float32),
        ],
    )(dom_x, w1cat, degp)


def _tc_layer2(agg1, p1, dinv, b1cat, w2cat):
    blk = 1000

    def body(a_ref, p1_ref, dinv_ref, b1_ref, w2_ref, r_ref):
        dv = dinv_ref[:, 0:1]
        h = jnp.maximum((a_ref[0] + a_ref[1] + p1_ref[...]) * dv + b1_ref[...], 0.0)
        r_ref[...] = jnp.dot(h * dv, w2_ref[...],
                             preferred_element_type=jnp.float32, precision=_HIGH)

    return pl.pallas_call(
        body,
        grid=(N_NODES // blk,),
        in_specs=[
            pl.BlockSpec((2, blk, 128), lambda i: (0, i, 0)),
            pl.BlockSpec((blk, 128), lambda i: (i, 0)),
            pl.BlockSpec((blk, 8), lambda i: (i, 0)),
            pl.BlockSpec((1, 128), lambda i: (0, 0)),
            pl.BlockSpec((128, 64), lambda i: (0, 0)),
        ],
        out_specs=pl.BlockSpec((blk, 64), lambda i: (i, 0)),
        out_shape=jax.ShapeDtypeStruct((N_NODES, 64), jnp.float32),
    )(agg1, p1, dinv, b1cat, w2cat)


def _tc_dom_pool(agg2, r, dinv, b2cat, batch3, pool_logs, cnt_logs):
    blk = 1000
    nblk = N_NODES // blk

    def body(a_ref, r_ref, dinv_ref, b2_ref, bat_ref, plg_ref, clg_ref,
             out_ref, pool_ref, cnt_ref):
        i = pl.program_id(0)
        z = ((a_ref[0] + a_ref[1] + r_ref[...]) * dinv_ref[:, 0:1]
             + b2_ref[...])
        b = bat_ref[0, 0, :]
        mask = (b[:, None] == lax.broadcasted_iota(jnp.int32, (1, N_GRAPHS), 1)
                ).astype(jnp.float32)
        psum = lax.dot_general(mask, z, (((0,), (0,)), ((), ())), precision=_HIGH)
        csum = lax.dot_general(mask, jnp.ones((blk, 8), jnp.float32),
                               (((0,), (0,)), ((), ())), precision=_HIGH)

        @pl.when(i == 0)
        def _():
            pool_ref[...] = jnp.zeros_like(pool_ref)
            cnt_ref[...] = jnp.zeros_like(cnt_ref)

        pool_ref[...] += psum
        cnt_ref[...] += csum

        @pl.when(i == nblk - 1)
        def _():
            md = pool_ref[...] / jnp.clip(cnt_ref[:, 0:1], 1.0)
            ml = plg_ref[...] / jnp.clip(clg_ref[:, 0:1], 1.0)
            dd = md[:, 32:64] - md[:, 0:32] + 1e-6
            dl = ml[:, 32:64] - ml[:, 0:32] + 1e-6
            out_ref[...] = (
                jnp.sqrt(jnp.sum(dd * dd, axis=1, keepdims=True))
                + jnp.sqrt(jnp.sum(dl * dl, axis=1, keepdims=True))
            )

    return pl.pallas_call(
        body,
        grid=(nblk,),
        in_specs=[
            pl.BlockSpec((2, blk, 64), lambda i: (0, i, 0)),
            pl.BlockSpec((blk, 64), lambda i: (i, 0)),
            pl.BlockSpec((blk, 8), lambda i: (i, 0)),
            pl.BlockSpec((1, 64), lambda i: (0, 0)),
            pl.BlockSpec((1, 1, blk), lambda i: (i, 0, 0)),
            pl.BlockSpec((N_GRAPHS, 64), lambda i: (0, 0)),
            pl.BlockSpec((N_GRAPHS, 8), lambda i: (0, 0)),
        ],
        out_specs=[
            pl.BlockSpec((N_GRAPHS, 1), lambda i: (0, 0)),
            pl.BlockSpec((N_GRAPHS, 64), lambda i: (0, 0)),
            pl.BlockSpec((N_GRAPHS, 8), lambda i: (0, 0)),
        ],
        out_shape=[
            jax.ShapeDtypeStruct((N_GRAPHS, 1), jnp.float32),
            jax.ShapeDtypeStruct((N_GRAPHS, 64), jnp.float32),
            jax.ShapeDtypeStruct((N_GRAPHS, 8), jnp.float32),
        ],
    )(agg2, r, dinv, b2cat, batch3, pool_logs, cnt_logs)


def _tc_logs_pool(logs_pad, wl1, bl1, wl2, bl2, lbatch3):
    blk = 1024
    n = 16384

    def body(x_ref, w1_ref, b1_ref, w2_ref, b2_ref, bat_ref, pool_ref, cnt_ref):
        i = pl.program_id(0)
        h = jnp.maximum(
            jnp.dot(x_ref[...], w1_ref[...], preferred_element_type=jnp.float32,
                    precision=_HIGH) + b1_ref[...], 0.0)
        z = jnp.dot(h, w2_ref[...], preferred_element_type=jnp.float32,
                    precision=_HIGH) + b2_ref[...]
        b = bat_ref[0, 0, :]
        mask = (b[:, None] == lax.broadcasted_iota(jnp.int32, (1, N_GRAPHS), 1)
                ).astype(jnp.float32)
        psum = lax.dot_general(mask, z, (((0,), (0,)), ((), ())), precision=_HIGH)
        csum = lax.dot_general(mask, jnp.ones((blk, 8), jnp.float32),
                               (((0,), (0,)), ((), ())), precision=_HIGH)

        @pl.when(i == 0)
        def _():
            pool_ref[...] = jnp.zeros_like(pool_ref)
            cnt_ref[...] = jnp.zeros_like(cnt_ref)

        pool_ref[...] += psum
        cnt_ref[...] += csum

    return pl.pallas_call(
        body,
        grid=(n // blk,),
        in_specs=[
            pl.BlockSpec((blk, 64), lambda i: (i, 0)),
            pl.BlockSpec((64, 64), lambda i: (0, 0)),
            pl.BlockSpec((1, 64), lambda i: (0, 0)),
            pl.BlockSpec((64, 64), lambda i: (0, 0)),
            pl.BlockSpec((1, 64), lambda i: (0, 0)),
            pl.BlockSpec((1, 1, blk), lambda i: (i, 0, 0)),
        ],
        out_specs=[
            pl.BlockSpec((N_GRAPHS, 64), lambda i: (0, 0)),
            pl.BlockSpec((N_GRAPHS, 8), lambda i: (0, 0)),
        ],
        out_shape=[
            jax.ShapeDtypeStruct((N_GRAPHS, 64), jnp.float32),
            jax.ShapeDtypeStruct((N_GRAPHS, 8), jnp.float32),
        ],
    )(logs_pad, wl1, bl1, wl2, bl2, lbatch3)


def kernel(dom_x, dom_edge_index, dom_batch, logs_x, logs_batch,
           Wg1, bg1, Wg2, bg2, Wt1, bt1, Wt2, bt2,
           Wlg1, blg1, Wlg2, blg2, Wlt1, blt1, Wlt2, blt2):
    f32 = jnp.float32
    w1cat = jnp.concatenate([Wg1, Wt1], axis=1).astype(f32)
    b1cat = jnp.concatenate([bg1, bt1])[None, :].astype(f32)
    w2cat = jnp.zeros((128, 64), f32).at[0:64, 0:32].set(Wg2).at[64:128, 32:64].set(Wt2)
    b2cat = jnp.concatenate([bg2, bt2])[None, :].astype(f32)
    wl1 = jnp.zeros((64, 64), f32).at[0:50, 0:25].set(Wlg1).at[0:50, 25:50].set(Wlt1)
    bl1 = jnp.zeros((1, 64), f32).at[0, 0:25].set(blg1).at[0, 25:50].set(blt1)
    wl2 = jnp.zeros((64, 64), f32).at[0:25, 0:32].set(Wlg2).at[25:50, 32:64].set(Wlt2)
    bl2 = jnp.concatenate([blg2, blt2])[None, :].astype(f32)
    logs_pad = jnp.pad(logs_x.astype(f32), ((0, 0), (0, 14)))

    src_i = dom_edge_index[0].astype(jnp.int32)
    dst_i = dom_edge_index[1].astype(jnp.int32)
    src_flat = src_i
    dst_r = dst_i.reshape(32, CPW, CHUNK)
    nxb = 32 * CPW_B * CHUNK_B - N_EDGES
    pad_ids = lax.iota(jnp.int32, nxb)
    src_flat_b = jnp.concatenate([src_i, pad_ids % N_NODES])
    dst_rb = jnp.concatenate(
        [dst_i, N_NODES + pad_ids % (N_PAD - N_NODES)]
    ).reshape(32, CPW_B, CHUNK_B)
    batch3 = dom_batch.astype(jnp.int32).reshape(10, 1, 1000)
    lbatch3 = logs_batch.astype(jnp.int32).reshape(16, 1, 1024)

    ones_rows = jnp.ones((CHUNK_B, 16), f32)
    z16 = jnp.zeros((ROWS_PER_SUB, 16), f32)
    z128 = jnp.zeros((ROWS_PER_SUB, 128), f32)
    z64 = jnp.zeros((ROWS_PER_SUB, 64), f32)

    degp = _sc_hist(dst_rb, ones_rows, z16)
    p1, dinv = _tc_scale1(dom_x.astype(f32), w1cat, degp)
    agg1 = _sc_agg(p1, src_flat, dst_r, z128, 128, CPW, CHUNK)
    pool_logs, cnt_logs = _tc_logs_pool(logs_pad, wl1, bl1, wl2, bl2, lbatch3)
    r = _tc_layer2(agg1, p1, dinv, b1cat, w2cat)
    agg2 = _sc_agg(r, src_flat_b, dst_rb, z64, 64, CPW_B, CHUNK_B)
    out, _, _ = _tc_dom_pool(agg2, r, dinv, b2cat, batch3, pool_logs, cnt_logs)
    return out.reshape(N_GRAPHS)


# speedup vs baseline: 1.0114x; 1.0114x over previous
import jax
import jax.numpy as jnp
from jax import lax
from jax.experimental import pallas as pl
from jax.experimental.pallas import tpu as pltpu, tpu_sc as plsc

N_NODES = 10000
N_PAD = 10240
N_EDGES = 320000
CHUNK = 80
CPW = 125
CHUNK_B = 128
CPW_B = 79
E_PAD = 32 * CPW * CHUNK
N_GRAPHS = 64
ROWS_PER_SUB = N_PAD // 16

_HIGH = lax.Precision.HIGHEST


def _sc_mesh():
    return plsc.VectorSubcoreMesh(
        core_axis_name="c", subcore_axis_name="s", num_cores=2, num_subcores=16
    )


def _sc_hist(dst_r, ones_rows, zrows):
    def body(dst_hbm, ones_hbm, z_hbm, out_hbm, dst_v, ones_v, agg_sh, sem):
        c = lax.axis_index("c")
        s = lax.axis_index("s")
        w = s * 2 + c
        pltpu.sync_copy(dst_hbm.at[w], dst_v)
        pltpu.sync_copy(ones_hbm, ones_v)
        pltpu.sync_copy(z_hbm, agg_sh.at[pl.ds(s * ROWS_PER_SUB, ROWS_PER_SUB)])
        plsc.subcore_barrier()

        def chunk(i, carry):
            pltpu.sync_copy(ones_v, agg_sh.at[dst_v.at[i]], add=True)
            return carry

        lax.fori_loop(0, CPW_B, chunk, 0)
        plsc.subcore_barrier()
        pltpu.sync_copy(
            agg_sh.at[pl.ds(s * ROWS_PER_SUB, ROWS_PER_SUB)],
            out_hbm.at[c, pl.ds(s * ROWS_PER_SUB, ROWS_PER_SUB)],
        )

    k = pl.kernel(
        body,
        out_type=jax.ShapeDtypeStruct((2, N_PAD, 16), jnp.float32),
        mesh=_sc_mesh(),
        compiler_params=pltpu.CompilerParams(use_tc_tiling_on_sc=False),
        scratch_types=[
            pltpu.VMEM((CPW_B, CHUNK_B), jnp.int32),
            pltpu.VMEM((CHUNK_B, 16), jnp.float32),
            pltpu.VMEM_SHARED((N_PAD, 16), jnp.float32),
            pltpu.SemaphoreType.DMA,
        ],
    )
    return k(dst_r, ones_rows, zrows)


def _sc_agg(p, src_r, dst_r, zrows, feat, cpw, chunk_sz):
    epw = cpw * chunk_sz

    def body(p_hbm, src_hbm, dst_hbm, z_hbm, out_hbm, src_v, dst_v, rows_a,
             rows_b, agg_sh, gsem_a, gsem_b, ssem_a, ssem_b):
        c = lax.axis_index("c")
        s = lax.axis_index("s")
        w = s * 2 + c
        pltpu.sync_copy(src_hbm.at[pl.ds(w * epw, epw)], src_v)
        pltpu.sync_copy(dst_hbm.at[w], dst_v)
        pltpu.sync_copy(z_hbm, agg_sh.at[pl.ds(s * ROWS_PER_SUB, ROWS_PER_SUB)])
        plsc.subcore_barrier()

        def g_start(i, buf, sem):
            pltpu.async_copy(p_hbm.at[src_v.at[pl.ds(i * chunk_sz, chunk_sz)]], buf, sem)

        def g_wait(i, buf, sem):
            pltpu.make_async_copy(
                p_hbm.at[src_v.at[pl.ds(i * chunk_sz, chunk_sz)]], buf, sem
            ).wait()

        def s_start(i, buf, sem):
            pltpu.async_copy(buf, agg_sh.at[dst_v.at[i]], sem, add=True)

        def s_wait(i, buf, sem):
            pltpu.make_async_copy(buf, agg_sh.at[dst_v.at[i]], sem).wait()

        g_start(0, rows_a, gsem_a)
        g_start(1, rows_b, gsem_b)

        def step(k, carry):
            i = 2 * k
            g_wait(i, rows_a, gsem_a)
            s_start(i, rows_a, ssem_a)
            g_wait(i + 1, rows_b, gsem_b)
            s_start(i + 1, rows_b, ssem_b)
            s_wait(i, rows_a, ssem_a)
            g_start(i + 2, rows_a, gsem_a)
            s_wait(i + 1, rows_b, ssem_b)

            @pl.when(i + 3 < cpw)
            def _():
                g_start(i + 3, rows_b, gsem_b)

            return carry

        lax.fori_loop(0, (cpw - 1) // 2, step, 0)
        g_wait(cpw - 1, rows_a, gsem_a)
        s_start(cpw - 1, rows_a, ssem_a)
        s_wait(cpw - 1, rows_a, ssem_a)
        plsc.subcore_barrier()
        pltpu.sync_copy(
            agg_sh.at[pl.ds(s * ROWS_PER_SUB, ROWS_PER_SUB)],
            out_hbm.at[c, pl.ds(s * ROWS_PER_SUB, ROWS_PER_SUB)],
        )

    extra = {} if feat == 128 else dict(
        compiler_params=pltpu.CompilerParams(use_tc_tiling_on_sc=False))
    k = pl.kernel(
        body,
        out_type=jax.ShapeDtypeStruct((2, N_PAD, feat), jnp.float32),
        mesh=_sc_mesh(),
        **extra,
        scratch_types=[
            pltpu.VMEM((cpw * chunk_sz,), jnp.int32),
            pltpu.VMEM((cpw, chunk_sz), jnp.int32),
            pltpu.VMEM((chunk_sz, feat), jnp.float32),
            pltpu.VMEM((chunk_sz, feat), jnp.float32),
            pltpu.VMEM_SHARED((N_PAD, feat), jnp.float32),
            pltpu.SemaphoreType.DMA,
            pltpu.SemaphoreType.DMA,
            pltpu.SemaphoreType.DMA,
            pltpu.SemaphoreType.DMA,
        ],
    )
    return k(p, src_r, dst_r, zrows)


def _tc_mm1(dom_x, w1cat):
    blk = 1000

    def body(x_ref, w_ref, xw_ref):
        xw_ref[...] = jnp.dot(x_ref[...], w_ref[...],
                              preferred_element_type=jnp.float32,
                              precision=_HIGH)

    return pl.pallas_call(
        body,
        grid=(N_NODES // blk,),
        in_specs=[
            pl.BlockSpec((blk, 128), lambda i: (i, 0)),
            pl.BlockSpec((128, 128), lambda i: (0, 0)),
        ],
        out_specs=pl.BlockSpec((blk, 128), lambda i: (i, 0)),
        out_shape=jax.ShapeDtypeStruct((N_NODES, 128), jnp.float32),
    )(dom_x, w1cat)


def _tc_scale1(xw, degp):
    blk = 1000

    def body(xw_ref, deg_ref, p1_ref, dinv_ref):
        d = deg_ref[0, :, 0:1] + deg_ref[1, :, 0:1] + 1.0
        dinv = lax.rsqrt(d)
        p1_ref[...] = xw_ref[...] * dinv
        dinv_ref[...] = jnp.broadcast_to(dinv, (blk, 8))

    return pl.pallas_call(
        body,
        grid=(N_NODES // blk,),
        in_specs=[
            pl.BlockSpec((blk, 128), lambda i: (i, 0)),
            pl.BlockSpec((2, blk, 16), lambda i: (0, i, 0)),
        ],
        out_specs=[
            pl.BlockSpec((blk, 128), lambda i: (i, 0)),
            pl.BlockSpec((blk, 8), lambda i: (i, 0)),
        ],
        out_shape=[
            jax.ShapeDtypeStruct((N_NODES, 128), jnp.float32),
            jax.ShapeDtypeStruct((N_NODES, 8), jnp.float32),
        ],
    )(xw, degp)


def _tc_layer2(agg1, p1, dinv, b1cat, w2cat):
    blk = 1000

    def body(a_ref, p1_ref, dinv_ref, b1_ref, w2_ref, r_ref):
        dv = dinv_ref[:, 0:1]
        h = jnp.maximum((a_ref[0] + a_ref[1] + p1_ref[...]) * dv + b1_ref[...], 0.0)
        r_ref[...] = jnp.dot(h * dv, w2_ref[...],
                             preferred_element_type=jnp.float32, precision=_HIGH)

    return pl.pallas_call(
        body,
        grid=(N_NODES // blk,),
        in_specs=[
            pl.BlockSpec((2, blk, 128), lambda i: (0, i, 0)),
            pl.BlockSpec((blk, 128), lambda i: (i, 0)),
            pl.BlockSpec((blk, 8), lambda i: (i, 0)),
            pl.BlockSpec((1, 128), lambda i: (0, 0)),
            pl.BlockSpec((128, 64), lambda i: (0, 0)),
        ],
        out_specs=pl.BlockSpec((blk, 64), lambda i: (i, 0)),
        out_shape=jax.ShapeDtypeStruct((N_NODES, 64), jnp.float32),
    )(agg1, p1, dinv, b1cat, w2cat)


def _tc_dom_pool(agg2, r, dinv, b2cat, batch3, pool_logs, cnt_logs):
    blk = 1000
    nblk = N_NODES // blk

    def body(a_ref, r_ref, dinv_ref, b2_ref, bat_ref, plg_ref, clg_ref,
             out_ref, pool_ref, cnt_ref):
        i = pl.program_id(0)
        z = ((a_ref[0] + a_ref[1] + r_ref[...]) * dinv_ref[:, 0:1]
             + b2_ref[...])
        b = bat_ref[0, 0, :]
        mask = (b[:, None] == lax.broadcasted_iota(jnp.int32, (1, N_GRAPHS), 1)
                ).astype(jnp.float32)
        psum = lax.dot_general(mask, z, (((0,), (0,)), ((), ())), precision=_HIGH)
        csum = lax.dot_general(mask, jnp.ones((blk, 8), jnp.float32),
                               (((0,), (0,)), ((), ())), precision=_HIGH)

        @pl.when(i == 0)
        def _():
            pool_ref[...] = jnp.zeros_like(pool_ref)
            cnt_ref[...] = jnp.zeros_like(cnt_ref)

        pool_ref[...] += psum
        cnt_ref[...] += csum

        @pl.when(i == nblk - 1)
        def _():
            md = pool_ref[...] / jnp.clip(cnt_ref[:, 0:1], 1.0)
            ml = plg_ref[...] / jnp.clip(clg_ref[:, 0:1], 1.0)
            dd = md[:, 32:64] - md[:, 0:32] + 1e-6
            dl = ml[:, 32:64] - ml[:, 0:32] + 1e-6
            out_ref[...] = (
                jnp.sqrt(jnp.sum(dd * dd, axis=1, keepdims=True))
                + jnp.sqrt(jnp.sum(dl * dl, axis=1, keepdims=True))
            )

    return pl.pallas_call(
        body,
        grid=(nblk,),
        in_specs=[
            pl.BlockSpec((2, blk, 64), lambda i: (0, i, 0)),
            pl.BlockSpec((blk, 64), lambda i: (i, 0)),
            pl.BlockSpec((blk, 8), lambda i: (i, 0)),
            pl.BlockSpec((1, 64), lambda i: (0, 0)),
            pl.BlockSpec((1, 1, blk), lambda i: (i, 0, 0)),
            pl.BlockSpec((N_GRAPHS, 64), lambda i: (0, 0)),
            pl.BlockSpec((N_GRAPHS, 8), lambda i: (0, 0)),
        ],
        out_specs=[
            pl.BlockSpec((N_GRAPHS, 1), lambda i: (0, 0)),
            pl.BlockSpec((N_GRAPHS, 64), lambda i: (0, 0)),
            pl.BlockSpec((N_GRAPHS, 8), lambda i: (0, 0)),
        ],
        out_shape=[
            jax.ShapeDtypeStruct((N_GRAPHS, 1), jnp.float32),
            jax.ShapeDtypeStruct((N_GRAPHS, 64), jnp.float32),
            jax.ShapeDtypeStruct((N_GRAPHS, 8), jnp.float32),
        ],
    )(agg2, r, dinv, b2cat, batch3, pool_logs, cnt_logs)


def _tc_logs_pool(logs_pad, wl1, bl1, wl2, bl2, lbatch3):
    blk = 1024
    n = 16384

    def body(x_ref, w1_ref, b1_ref, w2_ref, b2_ref, bat_ref, pool_ref, cnt_ref):
        i = pl.program_id(0)
        h = jnp.maximum(
            jnp.dot(x_ref[...], w1_ref[...], preferred_element_type=jnp.float32,
                    precision=_HIGH) + b1_ref[...], 0.0)
        z = jnp.dot(h, w2_ref[...], preferred_element_type=jnp.float32,
                    precision=_HIGH) + b2_ref[...]
        b = bat_ref[0, 0, :]
        mask = (b[:, None] == lax.broadcasted_iota(jnp.int32, (1, N_GRAPHS), 1)
                ).astype(jnp.float32)
        psum = lax.dot_general(mask, z, (((0,), (0,)), ((), ())), precision=_HIGH)
        csum = lax.dot_general(mask, jnp.ones((blk, 8), jnp.float32),
                               (((0,), (0,)), ((), ())), precision=_HIGH)

        @pl.when(i == 0)
        def _():
            pool_ref[...] = jnp.zeros_like(pool_ref)
            cnt_ref[...] = jnp.zeros_like(cnt_ref)

        pool_ref[...] += psum
        cnt_ref[...] += csum

    return pl.pallas_call(
        body,
        grid=(n // blk,),
        in_specs=[
            pl.BlockSpec((blk, 64), lambda i: (i, 0)),
            pl.BlockSpec((64, 64), lambda i: (0, 0)),
            pl.BlockSpec((1, 64), lambda i: (0, 0)),
            pl.BlockSpec((64, 64), lambda i: (0, 0)),
            pl.BlockSpec((1, 64), lambda i: (0, 0)),
            pl.BlockSpec((1, 1, blk), lambda i: (i, 0, 0)),
        ],
        out_specs=[
            pl.BlockSpec((N_GRAPHS, 64), lambda i: (0, 0)),
            pl.BlockSpec((N_GRAPHS, 8), lambda i: (0, 0)),
        ],
        out_shape=[
            jax.ShapeDtypeStruct((N_GRAPHS, 64), jnp.float32),
            jax.ShapeDtypeStruct((N_GRAPHS, 8), jnp.float32),
        ],
    )(logs_pad, wl1, bl1, wl2, bl2, lbatch3)


def kernel(dom_x, dom_edge_index, dom_batch, logs_x, logs_batch,
           Wg1, bg1, Wg2, bg2, Wt1, bt1, Wt2, bt2,
           Wlg1, blg1, Wlg2, blg2, Wlt1, blt1, Wlt2, blt2):
    f32 = jnp.float32
    w1cat = jnp.concatenate([Wg1, Wt1], axis=1).astype(f32)
    b1cat = jnp.concatenate([bg1, bt1])[None, :].astype(f32)
    w2cat = jnp.zeros((128, 64), f32).at[0:64, 0:32].set(Wg2).at[64:128, 32:64].set(Wt2)
    b2cat = jnp.concatenate([bg2, bt2])[None, :].astype(f32)
    wl1 = jnp.zeros((64, 64), f32).at[0:50, 0:25].set(Wlg1).at[0:50, 25:50].set(Wlt1)
    bl1 = jnp.zeros((1, 64), f32).at[0, 0:25].set(blg1).at[0, 25:50].set(blt1)
    wl2 = jnp.zeros((64, 64), f32).at[0:25, 0:32].set(Wlg2).at[25:50, 32:64].set(Wlt2)
    bl2 = jnp.concatenate([blg2, blt2])[None, :].astype(f32)
    logs_pad = jnp.pad(logs_x.astype(f32), ((0, 0), (0, 14)))

    src_i = dom_edge_index[0].astype(jnp.int32)
    dst_i = dom_edge_index[1].astype(jnp.int32)
    src_flat = src_i
    dst_r = dst_i.reshape(32, CPW, CHUNK)
    nxb = 32 * CPW_B * CHUNK_B - N_EDGES
    pad_ids = lax.iota(jnp.int32, nxb)
    src_flat_b = jnp.concatenate([src_i, pad_ids % N_NODES])
    dst_rb = jnp.concatenate(
        [dst_i, N_NODES + pad_ids % (N_PAD - N_NODES)]
    ).reshape(32, CPW_B, CHUNK_B)
    batch3 = dom_batch.astype(jnp.int32).reshape(10, 1, 1000)
    lbatch3 = logs_batch.astype(jnp.int32).reshape(16, 1, 1024)

    ones_rows = jnp.ones((CHUNK_B, 16), f32)
    z16 = jnp.zeros((ROWS_PER_SUB, 16), f32)
    z128 = jnp.zeros((ROWS_PER_SUB, 128), f32)
    z64 = jnp.zeros((ROWS_PER_SUB, 64), f32)

    xw = _tc_mm1(dom_x.astype(f32), w1cat)
    degp = _sc_hist(dst_rb, ones_rows, z16)
    p1, dinv = _tc_scale1(xw, degp)
    agg1 = _sc_agg(p1, src_flat, dst_r, z128, 128, CPW, CHUNK)
    pool_logs, cnt_logs = _tc_logs_pool(logs_pad, wl1, bl1, wl2, bl2, lbatch3)
    r = _tc_layer2(agg1, p1, dinv, b1cat, w2cat)
    agg2 = _sc_agg(r, src_flat_b, dst_rb, z64, 64, CPW_B, CHUNK_B)
    out, _, _ = _tc_dom_pool(agg2, r, dinv, b2cat, batch3, pool_logs, cnt_logs)
    return out.reshape(N_GRAPHS)
